# baseline (speedup 1.0000x reference)
import functools
import jax
import jax.numpy as jnp
from jax import lax
from jax.experimental import pallas as pl
from jax.experimental.pallas import tpu as pltpu
from jax.experimental.pallas import tpu_sc as plsc

M = 1_000_000
D = 64
B = 16384

NC = 2
NS = 16
HALF = M // NC
TROWS = M // (NC * NS)
IPT = B // NS
NV = IPT // 16
CH = 128
NCHUNK = IPT // CH
IMAX = jnp.int32(2147483647)


def _sc_body(t_hbm, idx_hbm, vrows_hbm, out_hbm, idx_v, idxbuf_v, vrows_v, sem):
    cid = lax.axis_index("c")
    sid = lax.axis_index("s")

    row0 = (cid * NS + sid) * TROWS
    pltpu.sync_copy(t_hbm.at[pl.ds(row0, TROWS)], out_hbm.at[pl.ds(row0, TROWS)])

    pltpu.sync_copy(idx_hbm.at[sid], idx_v)
    pltpu.sync_copy(vrows_hbm, vrows_v)

    plsc.subcore_barrier()

    lo = cid * HALF
    hi = lo + HALF

    best = IMAX
    for k in range(NV):
        iv = idx_v[pl.ds(k * 16, 16)]
        in_half = (iv >= lo) & (iv < hi)
        best = jnp.minimum(best, jnp.min(jnp.where(in_half, iv, IMAX)))

    for k in range(NV):
        iv = idx_v[pl.ds(k * 16, 16)]
        in_half = (iv >= lo) & (iv < hi)
        ivc = jnp.where(in_half, iv, best)
        idxbuf_v[k // (CH // 16), pl.ds((k % (CH // 16)) * 16, 16)] = ivc

    @pl.when(best != IMAX)
    def _():
        for c in range(NCHUNK):
            pltpu.async_copy(vrows_v, out_hbm.at[idxbuf_v.at[c]], sem).wait()


def kernel(t, idx, v):
    idx2 = idx.astype(jnp.int32).reshape(NS, IPT)
    vrows = jnp.full((CH, D), v, dtype=jnp.float32)

    fill = pl.kernel(
        _sc_body,
        out_type=jax.ShapeDtypeStruct((M, D), jnp.float32),
        mesh=plsc.VectorSubcoreMesh(core_axis_name="c", subcore_axis_name="s"),
        scratch_types=[
            pltpu.VMEM((IPT,), jnp.int32),
            pltpu.VMEM((NCHUNK, CH), jnp.int32),
            pltpu.VMEM((CH, D), jnp.float32),
            pltpu.SemaphoreType.DMA,
        ],
        compiler_params=pltpu.CompilerParams(
            use_tc_tiling_on_sc=False, needs_layout_passes=False
        ),
    )
    return fill(t, idx2, vrows)


# speedup vs baseline: 6.3883x; 6.3883x over previous
import functools
import jax
import jax.numpy as jnp
from jax import lax
from jax.experimental import pallas as pl
from jax.experimental.pallas import tpu as pltpu
from jax.experimental.pallas import tpu_sc as plsc

M = 1_000_000
D = 64
B = 16384

NC = 2
NS = 16
HALF = M // NC
TROWS = M // (NC * NS)
IPT = B // NS
NV = IPT // 16
CH = 128
NCHUNK = IPT // CH
IMAX = jnp.int32(2147483647)


CROWS = 625
NCHK = TROWS // CROWS


def _sc_body(t_hbm, idx_hbm, vrows_hbm, out_hbm, idx_v, idxbuf_v, vrows_v,
             cbuf0, cbuf1, in_sem, out_sem, sem):
    cid = lax.axis_index("c")
    sid = lax.axis_index("s")

    row0 = (cid * NS + sid) * TROWS
    cbuf = [cbuf0, cbuf1]

    def chunk(i):
        return pl.ds(row0 + i * CROWS, CROWS)

    din = [None, None]
    dout = [None, None]
    din[0] = pltpu.async_copy(t_hbm.at[chunk(0)], cbuf[0], in_sem)
    for i in range(NCHK):
        b = i & 1
        if i + 1 < NCHK:
            if dout[1 - b] is not None:
                dout[1 - b].wait()
            din[1 - b] = pltpu.async_copy(t_hbm.at[chunk(i + 1)], cbuf[1 - b],
                                          in_sem)
        din[b].wait()
        dout[b] = pltpu.async_copy(cbuf[b], out_hbm.at[chunk(i)], out_sem)

    pltpu.sync_copy(idx_hbm.at[sid], idx_v)
    pltpu.sync_copy(vrows_hbm, vrows_v)

    dout[0].wait()
    dout[1].wait()

    plsc.subcore_barrier()

    lo = cid * HALF
    hi = lo + HALF

    best = IMAX
    for k in range(NV):
        iv = idx_v[pl.ds(k * 16, 16)]
        in_half = (iv >= lo) & (iv < hi)
        best = jnp.minimum(best, jnp.min(jnp.where(in_half, iv, IMAX)))

    for k in range(NV):
        iv = idx_v[pl.ds(k * 16, 16)]
        in_half = (iv >= lo) & (iv < hi)
        ivc = jnp.where(in_half, iv, best)
        idxbuf_v[k // (CH // 16), pl.ds((k % (CH // 16)) * 16, 16)] = ivc

    @pl.when(best != IMAX)
    def _():
        for c in range(NCHUNK):
            pltpu.async_copy(vrows_v, out_hbm.at[idxbuf_v.at[c]], sem).wait()


def kernel(t, idx, v):
    idx2 = idx.astype(jnp.int32).reshape(NS, IPT)
    vrows = jnp.full((CH, D), v, dtype=jnp.float32)

    fill = pl.kernel(
        _sc_body,
        out_type=jax.ShapeDtypeStruct((M, D), jnp.float32),
        mesh=plsc.VectorSubcoreMesh(core_axis_name="c", subcore_axis_name="s"),
        scratch_types=[
            pltpu.VMEM((IPT,), jnp.int32),
            pltpu.VMEM((NCHUNK, CH), jnp.int32),
            pltpu.VMEM((CH, D), jnp.float32),
            pltpu.VMEM((CROWS, D), jnp.float32),
            pltpu.VMEM((CROWS, D), jnp.float32),
            pltpu.SemaphoreType.DMA,
            pltpu.SemaphoreType.DMA,
            pltpu.SemaphoreType.DMA,
        ],
        compiler_params=pltpu.CompilerParams(
            use_tc_tiling_on_sc=False, needs_layout_passes=False
        ),
    )
    return fill(t, idx2, vrows)


# speedup vs baseline: 40.6475x; 6.3628x over previous
import jax
import jax.numpy as jnp
from jax import lax
from jax.experimental import pallas as pl
from jax.experimental.pallas import tpu as pltpu
from jax.experimental.pallas import tpu_sc as plsc

M = 1_000_000
D = 64
B = 16384

NC = 2
NS = 16
IPT = B // NS
RP = 31264
HALFP = NS * RP
MP = NC * HALFP
SH = HALFP + 256
ZP = SH // NS

CC = 8192
GRID = -(-M // CC)
MPAD = GRID * CC


def _mask_body(idx_hbm, zeros_hbm, mask_hbm, idx_v, idxc_v, ones_v, stage_v,
               shared, sem):
    cid = lax.axis_index("c")
    sid = lax.axis_index("s")

    z0 = pl.multiple_of(sid * ZP, 8)
    pltpu.sync_copy(zeros_hbm, stage_v)
    pltpu.sync_copy(stage_v, shared.at[pl.ds(z0, ZP)])
    pltpu.sync_copy(idx_hbm.at[sid], idx_v)
    for k in range(8):
        ones_v[pl.ds(k * 16, 16)] = jnp.ones((16,), jnp.float32)

    plsc.subcore_barrier()

    lo = cid * HALFP
    for k in range(IPT // 16):
        iv = idx_v[pl.ds(k * 16, 16)]
        loc = iv - lo
        ok = (loc >= 0) & (loc < HALFP)
        locc = jnp.where(ok, loc, jnp.int32(HALFP))
        idxc_v[k // 8, pl.ds((k % 8) * 16, 16)] = locc

    def _scatter():
        for j in range(IPT // 128):
            pltpu.sync_copy(ones_v, shared.at[idxc_v.at[j]], add=True)

    for turn in range(NS):
        pl.when(sid == turn)(_scatter)
        plsc.subcore_barrier()

    s0 = pl.multiple_of(sid * RP, 8)
    d0 = pl.multiple_of(cid * HALFP + sid * RP, 8)
    pltpu.sync_copy(shared.at[pl.ds(s0, RP)], stage_v.at[pl.ds(0, RP)])
    pltpu.sync_copy(stage_v.at[pl.ds(0, RP)], mask_hbm.at[pl.ds(d0, RP)])


def _select_body(v_sm, t_ref, m_ref, o_ref):
    m = m_ref[...]
    o_ref[...] = jnp.where(m[None, :] != 0.0, v_sm[0, 0], t_ref[...])


def kernel(t, idx, v):
    tT = t.T
    idx2 = idx.astype(jnp.int32).reshape(NS, IPT)
    zeros = jnp.zeros((ZP,), jnp.float32)
    v2 = jnp.reshape(v, (1, 1))

    build_mask = pl.kernel(
        _mask_body,
        out_type=jax.ShapeDtypeStruct((MPAD,), jnp.float32),
        mesh=plsc.VectorSubcoreMesh(core_axis_name="c", subcore_axis_name="s"),
        scratch_types=[
            pltpu.VMEM((IPT,), jnp.int32),
            pltpu.VMEM((IPT // 128, 128), jnp.int32),
            pltpu.VMEM((128,), jnp.float32),
            pltpu.VMEM((ZP,), jnp.float32),
            pltpu.VMEM_SHARED((SH,), jnp.float32),
            pltpu.SemaphoreType.DMA,
        ],
    )
    mask = build_mask(idx2, zeros)

    outT = pl.pallas_call(
        _select_body,
        grid=(GRID,),
        in_specs=[
            pl.BlockSpec(memory_space=pltpu.SMEM),
            pl.BlockSpec((D, CC), lambda i: (0, i)),
            pl.BlockSpec((CC,), lambda i: (i,)),
        ],
        out_specs=pl.BlockSpec((D, CC), lambda i: (0, i)),
        out_shape=jax.ShapeDtypeStruct((D, M), jnp.float32),
    )(v2, tT, mask)

    return outT.T
